# baseline (speedup 1.0000x reference)
import functools

import jax
import jax.numpy as jnp
from jax import lax
from jax.experimental import pallas as pl
from jax.experimental.pallas import tpu as pltpu
from jax.experimental.pallas import tpu_sc as plsc

_NC = 2
_NS = 16
_NW = _NC * _NS

_G = 128
_KG = 2


def _sc_gather(table, ids2d, n_rows, h):
    rows_per_w = n_rows // _NW
    rows_per_it = _KG * _G
    n_it = rows_per_w // rows_per_it
    assert rows_per_w % rows_per_it == 0
    id_rows_per_w = rows_per_w // _G

    mesh = plsc.VectorSubcoreMesh(
        core_axis_name="c", subcore_axis_name="s",
        num_cores=_NC, num_subcores=_NS)

    @functools.partial(
        pl.kernel,
        out_type=jax.ShapeDtypeStruct((n_rows, h), jnp.float32),
        mesh=mesh,
        scratch_types=[
            pltpu.VMEM((id_rows_per_w, _G), jnp.int32),
            pltpu.VMEM((2 * rows_per_it, h), jnp.float32),
            pltpu.SemaphoreType.DMA,
            pltpu.SemaphoreType.DMA,
        ],
    )
    def gather_kernel(table_hbm, ids_hbm, out_hbm, idx_v, rows_v, sem_g, sem_w):
        wid = lax.axis_index("s") * _NC + lax.axis_index("c")
        row0 = wid * rows_per_w

        pltpu.sync_copy(ids_hbm.at[wid], idx_v)

        def body(g, carry):
            half = (g % 2) * rows_per_it
            @pl.when(g >= 2)
            def _drain():
                pltpu.make_async_copy(
                    rows_v.at[pl.ds(half, rows_per_it)],
                    out_hbm.at[pl.ds(row0, rows_per_it)],
                    sem_w).wait()

            copies = [
                pltpu.async_copy(
                    table_hbm.at[idx_v.at[g * _KG + j]],
                    rows_v.at[pl.ds(half + j * _G, _G)],
                    sem_g)
                for j in range(_KG)
            ]
            for c in copies:
                c.wait()
            pltpu.async_copy(
                rows_v.at[pl.ds(half, rows_per_it)],
                out_hbm.at[pl.ds(row0 + g * rows_per_it, rows_per_it)],
                sem_w)
            return carry

        lax.fori_loop(0, n_it, body, 0)
        for _ in range(2):
            pltpu.make_async_copy(
                rows_v.at[pl.ds(0, rows_per_it)],
                out_hbm.at[pl.ds(row0, rows_per_it)],
                sem_w).wait()

    return gather_kernel(table, ids2d)


def _tc_body(resp_ref, e_ref, wi_ref, wr_ref, b_ref, out_ref, *, k):
    sb = out_ref.shape[0]
    c = lax.broadcasted_iota(jnp.int32, (k, 1), 0).astype(jnp.float32)
    inv = 1.0 / (k - 1)
    bias = b_ref[...]
    wi = wi_ref[...]
    wr = wr_ref[...]
    for s in range(sb):
        es = e_ref[s]
        acc = lax.dot_general(wi, es, (((1,), (1,)), ((), ())),
                              preferred_element_type=jnp.float32)
        rf = resp_ref[s, :].astype(jnp.float32).reshape(1, -1)
        w = jnp.maximum(1.0 - jnp.abs(c - rf) * inv, 0.0)
        acc = acc + lax.dot_general(wr, w, (((1,), (0,)), ((), ())),
                                    preferred_element_type=jnp.float32)
        out_ref[s] = acc + bias


def _tc_project_chunk(e3d, resp_t, w_item, w_resp_w, bias2d, s0, out_prev, sb):
    sc, b, h = e3d.shape
    s_tot = resp_t.shape[0]
    v = w_item.shape[0]
    k = w_resp_w.shape[1]
    grid = (sc // sb,)
    sb0 = s0 // sb
    in_specs = [
        pl.BlockSpec((sb, b), lambda i: (i + sb0, 0)),
        pl.BlockSpec((sb, b, h), lambda i: (i, 0, 0)),
        pl.BlockSpec((v, h), lambda i: (0, 0)),
        pl.BlockSpec((v, k), lambda i: (0, 0)),
        pl.BlockSpec((v, 1), lambda i: (0, 0)),
    ]
    args = [resp_t, e3d, w_item, w_resp_w, bias2d]
    aliases = {}
    if out_prev is not None:
        in_specs = [pl.BlockSpec(memory_space=pltpu.MemorySpace.HBM)] + in_specs
        args = [out_prev] + args
        aliases = {0: 0}

    def body(*refs):
        _tc_body(*refs[-6:], k=k)

    return pl.pallas_call(
        body,
        grid=grid,
        in_specs=in_specs,
        out_specs=pl.BlockSpec((sb, v, b), lambda i: (i + sb0, 0, 0)),
        out_shape=jax.ShapeDtypeStruct((s_tot, v, b), jnp.float32),
        input_output_aliases=aliases,
    )(*args)


def kernel(question_ids, responses, item_embed, W_item, W_resp_w, W_resp_b):
    b, s = question_ids.shape
    n = b * s
    q1, h = item_embed.shape
    v = W_item.shape[0]
    k = W_resp_w.shape[1]

    qt = question_ids.T.astype(jnp.int32)
    resp_t = responses.T.astype(jnp.int32)
    bias2d = W_resp_b.reshape(v, 1)
    bounds = [0, 32, 96, 160, s]
    out = None
    for c in range(len(bounds) - 1):
        s0, s1 = bounds[c], bounds[c + 1]
        nc = (s1 - s0) * b
        ids3d = qt[s0:s1].reshape(_NW, nc // (_NW * _G), _G)
        e_c = _sc_gather(item_embed, ids3d, nc, h)
        out = _tc_project_chunk(e_c.reshape(s1 - s0, b, h), resp_t,
                                W_item, W_resp_w, bias2d, s0, out, sb=8)
    return jnp.transpose(out, (2, 0, 1))


# speedup vs baseline: 1.0242x; 1.0242x over previous
import functools

import jax
import jax.numpy as jnp
from jax import lax
from jax.experimental import pallas as pl
from jax.experimental.pallas import tpu as pltpu
from jax.experimental.pallas import tpu_sc as plsc

_NC = 2
_NS = 16
_NW = _NC * _NS

_G = 128
_KG = 2


def _sc_gather(table, ids2d, n_rows, h):
    rows_per_w = n_rows // _NW
    rows_per_it = _KG * _G
    n_it = rows_per_w // rows_per_it
    assert rows_per_w % rows_per_it == 0
    id_rows_per_w = rows_per_w // _G

    mesh = plsc.VectorSubcoreMesh(
        core_axis_name="c", subcore_axis_name="s",
        num_cores=_NC, num_subcores=_NS)

    @functools.partial(
        pl.kernel,
        out_type=jax.ShapeDtypeStruct((n_rows, h), jnp.float32),
        mesh=mesh,
        scratch_types=[
            pltpu.VMEM((id_rows_per_w, _G), jnp.int32),
            pltpu.VMEM((2 * rows_per_it, h), jnp.float32),
            pltpu.SemaphoreType.DMA,
            pltpu.SemaphoreType.DMA,
        ],
    )
    def gather_kernel(table_hbm, ids_hbm, out_hbm, idx_v, rows_v, sem_g, sem_w):
        wid = lax.axis_index("s") * _NC + lax.axis_index("c")
        row0 = wid * rows_per_w

        pltpu.sync_copy(ids_hbm.at[wid], idx_v)

        def body(g, carry):
            half = (g % 2) * rows_per_it
            @pl.when(g >= 2)
            def _drain():
                pltpu.make_async_copy(
                    rows_v.at[pl.ds(half, rows_per_it)],
                    out_hbm.at[pl.ds(row0, rows_per_it)],
                    sem_w).wait()

            copies = [
                pltpu.async_copy(
                    table_hbm.at[idx_v.at[g * _KG + j]],
                    rows_v.at[pl.ds(half + j * _G, _G)],
                    sem_g)
                for j in range(_KG)
            ]
            for c in copies:
                c.wait()
            pltpu.async_copy(
                rows_v.at[pl.ds(half, rows_per_it)],
                out_hbm.at[pl.ds(row0 + g * rows_per_it, rows_per_it)],
                sem_w)
            return carry

        lax.fori_loop(0, n_it, body, 0)
        for _ in range(2):
            pltpu.make_async_copy(
                rows_v.at[pl.ds(0, rows_per_it)],
                out_hbm.at[pl.ds(row0, rows_per_it)],
                sem_w).wait()

    return gather_kernel(table, ids2d)


def _tc_body(resp_ref, e_ref, wi_ref, wr_ref, b_ref, out_ref, *, k):
    sb = out_ref.shape[0]
    c = lax.broadcasted_iota(jnp.int32, (k, 1), 0).astype(jnp.float32)
    inv = 1.0 / (k - 1)
    bias = b_ref[...]
    wi = wi_ref[...]
    wr = wr_ref[...]
    for s in range(sb):
        es = e_ref[s]
        acc = lax.dot_general(wi, es, (((1,), (1,)), ((), ())),
                              preferred_element_type=jnp.float32)
        rf = resp_ref[s, :].astype(jnp.float32).reshape(1, -1)
        w = jnp.maximum(1.0 - jnp.abs(c - rf) * inv, 0.0)
        acc = acc + lax.dot_general(wr, w, (((1,), (0,)), ((), ())),
                                    preferred_element_type=jnp.float32)
        out_ref[s] = acc + bias


def _tc_project_chunk(e3d, resp_t, w_item, w_resp_w, bias2d, s0, out_prev, sb):
    sc, b, h = e3d.shape
    s_tot = resp_t.shape[0]
    v = w_item.shape[0]
    k = w_resp_w.shape[1]
    grid = (sc // sb,)
    sb0 = s0 // sb
    in_specs = [
        pl.BlockSpec((sb, b), lambda i: (i + sb0, 0)),
        pl.BlockSpec((sb, b, h), lambda i: (i, 0, 0)),
        pl.BlockSpec((v, h), lambda i: (0, 0)),
        pl.BlockSpec((v, k), lambda i: (0, 0)),
        pl.BlockSpec((v, 1), lambda i: (0, 0)),
    ]
    args = [resp_t, e3d, w_item, w_resp_w, bias2d]
    aliases = {}
    if out_prev is not None:
        in_specs = [pl.BlockSpec(memory_space=pltpu.MemorySpace.HBM)] + in_specs
        args = [out_prev] + args
        aliases = {0: 0}

    def body(*refs):
        _tc_body(*refs[-6:], k=k)

    return pl.pallas_call(
        body,
        grid=grid,
        in_specs=in_specs,
        out_specs=pl.BlockSpec((sb, v, b), lambda i: (i + sb0, 0, 0)),
        out_shape=jax.ShapeDtypeStruct((s_tot, v, b), jnp.float32),
        input_output_aliases=aliases,
    )(*args)


def kernel(question_ids, responses, item_embed, W_item, W_resp_w, W_resp_b):
    b, s = question_ids.shape
    n = b * s
    q1, h = item_embed.shape
    v = W_item.shape[0]
    k = W_resp_w.shape[1]

    qt = question_ids.T.astype(jnp.int32)
    resp_t = responses.T.astype(jnp.int32)
    bias2d = W_resp_b.reshape(v, 1)
    bounds = [0, s]
    out = None
    for c in range(len(bounds) - 1):
        s0, s1 = bounds[c], bounds[c + 1]
        nc = (s1 - s0) * b
        ids3d = qt[s0:s1].reshape(_NW, nc // (_NW * _G), _G)
        e_c = _sc_gather(item_embed, ids3d, nc, h)
        out = _tc_project_chunk(e_c.reshape(s1 - s0, b, h), resp_t,
                                W_item, W_resp_w, bias2d, s0, out, sb=8)
    return jnp.transpose(out, (2, 0, 1))
